# baseline (speedup 1.0000x reference)
import jax
import jax.numpy as jnp
from jax.experimental import pallas as pl
from jax.experimental.pallas import tpu as pltpu

_EPS = 1e-8


def _knn_block_kernel(x_ref, w_ref, g_ref, o_ref, feat_ref, rmax_ref, ridx_ref,
                      *, kblk, k_total, nblk):
    i = pl.program_id(0)
    q = feat_ref.shape[0]

    @pl.when(i == 0)
    def _init():
        f = jnp.dot(x_ref[...], w_ref[...], preferred_element_type=jnp.float32)
        n = jnp.sqrt(jnp.sum(f * f, axis=1, keepdims=True))
        feat_ref[...] = f / jnp.maximum(n, _EPS)
        rmax_ref[...] = jnp.full(rmax_ref.shape, -jnp.inf, jnp.float32)
        ridx_ref[...] = jnp.zeros(ridx_ref.shape, jnp.int32)

    g = g_ref[...]
    gnorm = jnp.sqrt(jnp.sum(g * g, axis=1, keepdims=True))
    gn = g / jnp.maximum(gnorm, _EPS)
    sim = jax.lax.dot_general(
        feat_ref[...], gn, (((1,), (1,)), ((), ())),
        preferred_element_type=jnp.float32)
    col = jax.lax.broadcasted_iota(jnp.int32, (q, kblk), 1)
    sim = jnp.where(col < (k_total - i * kblk), sim, -jnp.inf)

    bmax = jnp.max(sim, axis=1, keepdims=True)
    barg = jnp.argmax(sim, axis=1).astype(jnp.int32).reshape(q, 1) + i * kblk
    better = bmax > rmax_ref[...]
    rmax_ref[...] = jnp.where(better, bmax, rmax_ref[...])
    ridx_ref[...] = jnp.where(better, barg, ridx_ref[...])

    @pl.when(i == nblk - 1)
    def _done():
        o_ref[...] = ridx_ref[...]


def kernel(x, W, training_features):
    q, d_in = x.shape
    d = W.shape[1]
    k_total = training_features.shape[0]
    kblk = 2048
    nblk = pl.cdiv(k_total, kblk)
    kpad = nblk * kblk
    g = jnp.pad(training_features, ((0, kpad - k_total), (0, 0)))

    import functools
    body = functools.partial(_knn_block_kernel, kblk=kblk, k_total=k_total,
                             nblk=nblk)
    out = pl.pallas_call(
        body,
        grid=(nblk,),
        in_specs=[
            pl.BlockSpec((q, d_in), lambda i: (0, 0)),
            pl.BlockSpec((d_in, d), lambda i: (0, 0)),
            pl.BlockSpec((kblk, d), lambda i: (i, 0)),
        ],
        out_specs=pl.BlockSpec((q, 1), lambda i: (0, 0)),
        out_shape=jax.ShapeDtypeStruct((q, 1), jnp.int32),
        scratch_shapes=[
            pltpu.VMEM((q, d), jnp.float32),
            pltpu.VMEM((q, 1), jnp.float32),
            pltpu.VMEM((q, 1), jnp.int32),
        ],
    )(x, W, g)
    return out.reshape(q)


# speedup vs baseline: 1.2775x; 1.2775x over previous
import jax
import jax.numpy as jnp
from jax.experimental import pallas as pl
from jax.experimental.pallas import tpu as pltpu

_EPS = 1e-8


def _knn_block_kernel(x_ref, w_ref, g_ref, o_ref, feat_ref, rmax_ref, ridx_ref,
                      *, kblk, nblk):
    i = pl.program_id(0)
    q = feat_ref.shape[0]

    @pl.when(i == 0)
    def _init():
        f = jnp.dot(x_ref[...], w_ref[...], preferred_element_type=jnp.float32)
        n = jnp.sqrt(jnp.sum(f * f, axis=1, keepdims=True))
        feat_ref[...] = f / jnp.maximum(n, _EPS)
        rmax_ref[...] = jnp.full(rmax_ref.shape, -jnp.inf, jnp.float32)
        ridx_ref[...] = jnp.zeros(ridx_ref.shape, jnp.int32)

    g = g_ref[...]
    gnorm = jnp.sqrt(jnp.sum(g * g, axis=1, keepdims=True))
    gn = g / jnp.maximum(gnorm, _EPS)
    sim = jax.lax.dot_general(
        feat_ref[...], gn, (((1,), (1,)), ((), ())),
        preferred_element_type=jnp.float32)

    bmax = jnp.max(sim, axis=1, keepdims=True)
    barg = jnp.argmax(sim, axis=1).astype(jnp.int32).reshape(q, 1) + i * kblk
    better = bmax > rmax_ref[...]
    rmax_ref[...] = jnp.where(better, bmax, rmax_ref[...])
    ridx_ref[...] = jnp.where(better, barg, ridx_ref[...])

    @pl.when(i == nblk - 1)
    def _done():
        o_ref[...] = ridx_ref[...]


def kernel(x, W, training_features):
    q, d_in = x.shape
    d = W.shape[1]
    k_total = training_features.shape[0]
    kblk = 4096
    nblk = pl.cdiv(k_total, kblk)
    kpad = nblk * kblk
    pad_rows = jnp.broadcast_to(training_features[:1], (kpad - k_total, d))
    g = jnp.concatenate([training_features, pad_rows], axis=0)

    import functools
    body = functools.partial(_knn_block_kernel, kblk=kblk, nblk=nblk)
    out = pl.pallas_call(
        body,
        grid=(nblk,),
        in_specs=[
            pl.BlockSpec((q, d_in), lambda i: (0, 0)),
            pl.BlockSpec((d_in, d), lambda i: (0, 0)),
            pl.BlockSpec((kblk, d), lambda i: (i, 0)),
        ],
        out_specs=pl.BlockSpec((q, 1), lambda i: (0, 0)),
        out_shape=jax.ShapeDtypeStruct((q, 1), jnp.int32),
        scratch_shapes=[
            pltpu.VMEM((q, d), jnp.float32),
            pltpu.VMEM((q, 1), jnp.float32),
            pltpu.VMEM((q, 1), jnp.int32),
        ],
    )(x, W, g)
    return out.reshape(q)


# speedup vs baseline: 1.4047x; 1.0996x over previous
import jax
import jax.numpy as jnp
from jax.experimental import pallas as pl
from jax.experimental.pallas import tpu as pltpu

_EPS = 1e-8


def _knn_block_kernel(x_ref, w_ref, g_ref, o_ref, feat_ref, rmax_ref, ridx_ref,
                      *, kblk, nblk):
    i = pl.program_id(0)
    q = feat_ref.shape[0]

    @pl.when(i == 0)
    def _init():
        f = jnp.dot(x_ref[...], w_ref[...], preferred_element_type=jnp.float32)
        n = jnp.sqrt(jnp.sum(f * f, axis=1, keepdims=True))
        feat_ref[...] = f / jnp.maximum(n, _EPS)
        rmax_ref[...] = jnp.full(rmax_ref.shape, -jnp.inf, jnp.float32)
        ridx_ref[...] = jnp.zeros(ridx_ref.shape, jnp.int32)

    g = g_ref[...]
    gss = jnp.sum(g * g, axis=1, keepdims=True)
    gn = g * jax.lax.rsqrt(jnp.maximum(gss, _EPS * _EPS))
    sim = jax.lax.dot_general(
        feat_ref[...], gn, (((1,), (1,)), ((), ())),
        preferred_element_type=jnp.float32)

    bmax = jnp.max(sim, axis=1, keepdims=True)
    rev = jax.lax.broadcasted_iota(jnp.int32, (q, kblk), 1) ^ (kblk - 1)
    hit = jnp.where(sim >= bmax, rev, 0)
    barg = (kblk - 1 - jnp.max(hit, axis=1, keepdims=True)) + i * kblk
    better = bmax > rmax_ref[...]
    rmax_ref[...] = jnp.where(better, bmax, rmax_ref[...])
    ridx_ref[...] = jnp.where(better, barg, ridx_ref[...])

    @pl.when(i == nblk - 1)
    def _done():
        o_ref[...] = ridx_ref[...]


def kernel(x, W, training_features):
    q, d_in = x.shape
    d = W.shape[1]
    k_total = training_features.shape[0]
    kblk = 4096
    nblk = pl.cdiv(k_total, kblk)
    kpad = nblk * kblk
    pad_rows = jnp.broadcast_to(training_features[:1], (kpad - k_total, d))
    g = jnp.concatenate([training_features, pad_rows], axis=0)

    import functools
    body = functools.partial(_knn_block_kernel, kblk=kblk, nblk=nblk)
    out = pl.pallas_call(
        body,
        grid=(nblk,),
        in_specs=[
            pl.BlockSpec((q, d_in), lambda i: (0, 0)),
            pl.BlockSpec((d_in, d), lambda i: (0, 0)),
            pl.BlockSpec((kblk, d), lambda i: (i, 0)),
        ],
        out_specs=pl.BlockSpec((q, 1), lambda i: (0, 0)),
        out_shape=jax.ShapeDtypeStruct((q, 1), jnp.int32),
        scratch_shapes=[
            pltpu.VMEM((q, d), jnp.float32),
            pltpu.VMEM((q, 1), jnp.float32),
            pltpu.VMEM((q, 1), jnp.int32),
        ],
    )(x, W, g)
    return out.reshape(q)


# speedup vs baseline: 1.5557x; 1.1075x over previous
import jax
import jax.numpy as jnp
from jax.experimental import pallas as pl
from jax.experimental.pallas import tpu as pltpu

_EPS = 1e-8


def _knn_block_kernel(x_ref, w_ref, g_ref, o_ref, feat_ref, rmax_ref, ridx_ref,
                      *, kblk, nblk):
    i = pl.program_id(0)
    q = feat_ref.shape[0]

    @pl.when(i == 0)
    def _init():
        f = jnp.dot(x_ref[...], w_ref[...], preferred_element_type=jnp.float32)
        n = jnp.sqrt(jnp.sum(f * f, axis=1, keepdims=True))
        feat_ref[...] = f / jnp.maximum(n, _EPS)
        rmax_ref[...] = jnp.full(rmax_ref.shape, -jnp.inf, jnp.float32)
        ridx_ref[...] = jnp.zeros(ridx_ref.shape, jnp.int32)

    g = g_ref[...]
    gss = jnp.sum(g * g, axis=1, keepdims=True)
    gn = g * jax.lax.rsqrt(jnp.maximum(gss, _EPS * _EPS))
    sim = jax.lax.dot_general(
        feat_ref[...], gn, (((1,), (1,)), ((), ())),
        preferred_element_type=jnp.float32)

    bmax = jnp.max(sim, axis=1, keepdims=True)
    rev = (jax.lax.broadcasted_iota(jnp.int32, (q, kblk), 1)
           ^ (kblk - 1)).astype(jnp.float32)
    hit = jnp.where(sim >= bmax, rev, 0.0)
    hmax = jnp.max(hit, axis=1, keepdims=True)
    barg = (kblk - 1 - hmax.astype(jnp.int32)) + i * kblk
    better = bmax > rmax_ref[...]
    rmax_ref[...] = jnp.where(better, bmax, rmax_ref[...])
    ridx_ref[...] = jnp.where(better, barg, ridx_ref[...])

    @pl.when(i == nblk - 1)
    def _done():
        o_ref[...] = ridx_ref[...]


def kernel(x, W, training_features):
    q, d_in = x.shape
    d = W.shape[1]
    k_total = training_features.shape[0]
    kblk = 4096
    nblk = pl.cdiv(k_total, kblk)
    kpad = nblk * kblk
    pad_rows = jnp.broadcast_to(training_features[:1], (kpad - k_total, d))
    g = jnp.concatenate([training_features, pad_rows], axis=0)

    import functools
    body = functools.partial(_knn_block_kernel, kblk=kblk, nblk=nblk)
    out = pl.pallas_call(
        body,
        grid=(nblk,),
        in_specs=[
            pl.BlockSpec((q, d_in), lambda i: (0, 0)),
            pl.BlockSpec((d_in, d), lambda i: (0, 0)),
            pl.BlockSpec((kblk, d), lambda i: (i, 0)),
        ],
        out_specs=pl.BlockSpec((q, 1), lambda i: (0, 0)),
        out_shape=jax.ShapeDtypeStruct((q, 1), jnp.int32),
        scratch_shapes=[
            pltpu.VMEM((q, d), jnp.float32),
            pltpu.VMEM((q, 1), jnp.float32),
            pltpu.VMEM((q, 1), jnp.int32),
        ],
    )(x, W, g)
    return out.reshape(q)


# speedup vs baseline: 1.9929x; 1.2810x over previous
import jax
import jax.numpy as jnp
from jax.experimental import pallas as pl
from jax.experimental.pallas import tpu as pltpu

_EPS = 1e-8


def _knn_block_kernel(x_ref, w_ref, g_ref, o_ref, feat_ref, rmax_ref, ridx_ref,
                      *, kblk, nblk):
    i = pl.program_id(0)
    q = feat_ref.shape[0]

    @pl.when(i == 0)
    def _init():
        f = jnp.dot(x_ref[...], w_ref[...], preferred_element_type=jnp.float32)
        n = jnp.sqrt(jnp.sum(f * f, axis=1, keepdims=True))
        feat_ref[...] = f / jnp.maximum(n, _EPS)
        rmax_ref[...] = jnp.full(rmax_ref.shape, -jnp.inf, jnp.float32)
        ridx_ref[...] = jnp.zeros(ridx_ref.shape, jnp.int32)

    g = g_ref[...]
    gss = jnp.sum(g * g, axis=1, keepdims=True)
    gn = g * jax.lax.rsqrt(jnp.maximum(gss, _EPS * _EPS))
    sim = jax.lax.dot_general(
        feat_ref[...], gn, (((1,), (1,)), ((), ())),
        preferred_element_type=jnp.float32)

    bmax = jnp.max(sim, axis=1, keepdims=True)
    rev = ((kblk - 1) - jax.lax.broadcasted_iota(jnp.int32, (q, kblk), 1)
           ).astype(jnp.float32)
    hit = jnp.where(sim >= bmax, rev, 0.0)
    hmax = jnp.max(hit, axis=1, keepdims=True)
    barg = (kblk - 1 - hmax.astype(jnp.int32)) + i * kblk
    better = bmax > rmax_ref[...]
    rmax_ref[...] = jnp.where(better, bmax, rmax_ref[...])
    ridx_ref[...] = jnp.where(better, barg, ridx_ref[...])

    @pl.when(i == nblk - 1)
    def _done():
        o_ref[...] = ridx_ref[...]


def kernel(x, W, training_features):
    q, d_in = x.shape
    d = W.shape[1]
    k_total = training_features.shape[0]
    kblk = 4000
    nblk = pl.cdiv(k_total, kblk)
    if k_total % kblk:
        pad_rows = jnp.broadcast_to(training_features[:1],
                                    (nblk * kblk - k_total, d))
        g = jnp.concatenate([training_features, pad_rows], axis=0)
    else:
        g = training_features

    import functools
    body = functools.partial(_knn_block_kernel, kblk=kblk, nblk=nblk)
    out = pl.pallas_call(
        body,
        grid=(nblk,),
        in_specs=[
            pl.BlockSpec((q, d_in), lambda i: (0, 0)),
            pl.BlockSpec((d_in, d), lambda i: (0, 0)),
            pl.BlockSpec((kblk, d), lambda i: (i, 0)),
        ],
        out_specs=pl.BlockSpec((q, 1), lambda i: (0, 0)),
        out_shape=jax.ShapeDtypeStruct((q, 1), jnp.int32),
        scratch_shapes=[
            pltpu.VMEM((q, d), jnp.float32),
            pltpu.VMEM((q, 1), jnp.float32),
            pltpu.VMEM((q, 1), jnp.int32),
        ],
    )(x, W, g)
    return out.reshape(q)
